# baseline (speedup 1.0000x reference)
import functools

import jax
import jax.numpy as jnp
from jax import lax
from jax.experimental import pallas as pl
from jax.experimental.pallas import tpu as pltpu
from jax.experimental.pallas import tpu_sc as plsc

VOCAB = 50257
D = 768
S = 1024
B = 4
TOK = B * S
NC, NS = 2, 16
NW = NC * NS
TPW = TOK // NW
WPB = S // TPW
C = 32
NCHUNK = TPW // C
NVEC = D // 16
NSLICE = 4
PRE = TPW // NS * NSLICE

_mesh = plsc.VectorSubcoreMesh(core_axis_name="c", subcore_axis_name="s")


@functools.partial(
    pl.kernel,
    mesh=_mesh,
    out_type=jax.ShapeDtypeStruct((TOK, D), jnp.float32),
    scratch_types=[
        pltpu.VMEM((TPW,), jnp.int32),
        pltpu.VMEM((C, D), jnp.float32),
        pltpu.VMEM((C, D), jnp.float32),
        pltpu.VMEM((C, D), jnp.float32),
        pltpu.VMEM((C, D), jnp.float32),
        pltpu.VMEM_SHARED((NSLICE * TPW, D), jnp.float32),
        pltpu.SemaphoreType.DMA,
        pltpu.SemaphoreType.DMA,
        pltpu.SemaphoreType.DMA,
        pltpu.SemaphoreType.DMA,
        pltpu.SemaphoreType.DMA,
        pltpu.SemaphoreType.DMA,
        pltpu.SemaphoreType.DMA,
    ],
)
def _embed(ids_hbm, wte_hbm, wpe_hbm, out_hbm,
           idx_v, r0, r1, p0, p1, wpe_sh, sg0, sg1, sp0, sp1, ss0, ss1, spre):
    rows = (r0, r1)
    pos = (p0, p1)
    sg = (sg0, sg1)
    sp = (sp0, sp1)
    ss = (ss0, ss1)
    s_idx = lax.axis_index("s")
    c_idx = lax.axis_index("c")
    wid = s_idx * NC + c_idx
    base = wid * TPW

    q_pre = lax.div(s_idx, NSLICE)
    sub = lax.rem(s_idx, NSLICE)
    src_row = (2 * q_pre + c_idx) * TPW + sub * PRE
    dst_row = q_pre * TPW + sub * PRE
    pre = pltpu.async_copy(
        wpe_hbm.at[pl.ds(src_row, PRE)], wpe_sh.at[pl.ds(dst_row, PRE)], spre)

    pltpu.sync_copy(
        ids_hbm.at[lax.div(wid, WPB), pl.ds(lax.rem(wid, WPB) * TPW, TPW)],
        idx_v)
    g_first = pltpu.async_copy(
        wte_hbm.at[idx_v.at[pl.ds(0, C)]], rows[0], sg[0])
    pre.wait()
    plsc.subcore_barrier()

    pos_base = lax.rem(s_idx, NSLICE) * TPW

    def start(ch, b):
        g = pltpu.async_copy(
            wte_hbm.at[idx_v.at[pl.ds(ch * C, C)]], rows[b], sg[b])
        p = pltpu.async_copy(
            wpe_sh.at[pl.ds(pos_base + ch * C, C)], pos[b], sp[b])
        return g, p

    p_first = pltpu.async_copy(
        wpe_sh.at[pl.ds(pos_base, C)], pos[0], sp[0])
    inflight = {0: (g_first, p_first)}
    store_h = [None, None]
    for ch in range(NCHUNK):
        b = ch % 2
        if ch + 1 < NCHUNK:
            if store_h[1 - b] is not None:
                store_h[1 - b].wait()
                store_h[1 - b] = None
            inflight[ch + 1] = start(ch + 1, 1 - b)
        g, p = inflight.pop(ch)
        g.wait()
        p.wait()

        def add_row(r, carry):
            for j0 in range(0, NVEC, 16):
                vals = [pos[b][r, pl.ds((j0 + k) * 16, 16)]
                        for k in range(16)]
                for k in range(16):
                    plsc.addupdate(
                        rows[b].at[r, pl.ds((j0 + k) * 16, 16)], vals[k])
            return carry

        lax.fori_loop(0, C, add_row, 0)
        store_h[b] = pltpu.async_copy(
            rows[b], out_hbm.at[pl.ds(base + ch * C, C)], ss[b])
    for h in store_h:
        if h is not None:
            h.wait()


def kernel(input_ids, wte, wpe):
    out = _embed(input_ids.astype(jnp.int32), wte, wpe)
    return out.reshape(input_ids.shape + (wpe.shape[1],))


# speedup vs baseline: 1.0552x; 1.0552x over previous
import functools

import jax
import jax.numpy as jnp
from jax import lax
from jax.experimental import pallas as pl
from jax.experimental.pallas import tpu as pltpu
from jax.experimental.pallas import tpu_sc as plsc

VOCAB = 50257
D = 768
S = 1024
B = 4
TOK = B * S
NC, NS = 2, 16
NW = NC * NS
PW = S // NW
NVEC = D // 16

_mesh = plsc.VectorSubcoreMesh(core_axis_name="c", subcore_axis_name="s")


@functools.partial(
    pl.kernel,
    mesh=_mesh,
    out_type=jax.ShapeDtypeStruct((TOK, D), jnp.float32),
    scratch_types=[
        pltpu.VMEM((B, PW), jnp.int32),
        pltpu.VMEM((PW, D), jnp.float32),
        pltpu.VMEM((PW, D), jnp.float32),
        pltpu.VMEM((PW, D), jnp.float32),
        pltpu.SemaphoreType.DMA,
        pltpu.SemaphoreType.DMA,
        pltpu.SemaphoreType.DMA,
        pltpu.SemaphoreType.DMA,
        pltpu.SemaphoreType.DMA,
        pltpu.SemaphoreType.DMA,
    ],
)
def _embed(ids_hbm, wte_hbm, wpe_hbm, out_hbm,
           idx_v, r0, r1, pos_v, sg0, sg1, ss0, ss1, spos, sidx):
    rows = (r0, r1)
    sg = (sg0, sg1)
    ss = (ss0, ss1)
    wid = lax.axis_index("s") * NC + lax.axis_index("c")
    p0 = wid * PW

    pre = pltpu.async_copy(wpe_hbm.at[pl.ds(p0, PW)], pos_v, spos)
    id_h = [pltpu.async_copy(ids_hbm.at[bb, pl.ds(p0, PW)], idx_v.at[bb],
                             sidx)
            for bb in range(B)]
    for h in id_h:
        h.wait()
    g_first = pltpu.async_copy(wte_hbm.at[idx_v.at[0]], rows[0], sg[0])
    pre.wait()

    inflight = {0: g_first}
    store_h = [None, None]
    for ch in range(B):
        b = ch % 2
        if ch + 1 < B:
            if store_h[1 - b] is not None:
                store_h[1 - b].wait()
                store_h[1 - b] = None
            inflight[ch + 1] = pltpu.async_copy(
                wte_hbm.at[idx_v.at[ch + 1]], rows[1 - b], sg[1 - b])
        inflight.pop(ch).wait()

        def add_row(r, carry):
            for j in range(NVEC):
                plsc.addupdate(rows[b].at[r, pl.ds(j * 16, 16)],
                               pos_v[r, pl.ds(j * 16, 16)])
            return carry

        lax.fori_loop(0, PW, add_row, 0)
        store_h[b] = pltpu.async_copy(
            rows[b], out_hbm.at[pl.ds(ch * S + p0, PW)], ss[b])
    for h in store_h:
        if h is not None:
            h.wait()


def kernel(input_ids, wte, wpe):
    out = _embed(input_ids.astype(jnp.int32), wte, wpe)
    return out.reshape(input_ids.shape + (wpe.shape[1],))


# speedup vs baseline: 1.0593x; 1.0039x over previous
import functools

import jax
import jax.numpy as jnp
from jax import lax
from jax.experimental import pallas as pl
from jax.experimental.pallas import tpu as pltpu
from jax.experimental.pallas import tpu_sc as plsc

VOCAB = 50257
D = 768
S = 1024
B = 4
TOK = B * S
NC, NS = 2, 16
NW = NC * NS
PW = S // NW
NVEC = D // 16
NBUF = 3

_mesh = plsc.VectorSubcoreMesh(core_axis_name="c", subcore_axis_name="s")


@functools.partial(
    pl.kernel,
    mesh=_mesh,
    out_type=jax.ShapeDtypeStruct((TOK, D), jnp.float32),
    scratch_types=[
        pltpu.VMEM((B, PW), jnp.int32),
        pltpu.VMEM((PW, D), jnp.float32),
        pltpu.VMEM((PW, D), jnp.float32),
        pltpu.VMEM((PW, D), jnp.float32),
        pltpu.VMEM((PW, D), jnp.float32),
        pltpu.SemaphoreType.DMA,
        pltpu.SemaphoreType.DMA,
        pltpu.SemaphoreType.DMA,
        pltpu.SemaphoreType.DMA,
        pltpu.SemaphoreType.DMA,
        pltpu.SemaphoreType.DMA,
        pltpu.SemaphoreType.DMA,
        pltpu.SemaphoreType.DMA,
    ],
)
def _embed(ids_hbm, wte_hbm, wpe_hbm, out_hbm,
           idx_v, r0, r1, r2, pos_v,
           sg0, sg1, sg2, ss0, ss1, ss2, spos, sidx):
    rows = (r0, r1, r2)
    sg = (sg0, sg1, sg2)
    ss = (ss0, ss1, ss2)
    wid = lax.axis_index("s") * NC + lax.axis_index("c")
    p0 = wid * PW

    pre = pltpu.async_copy(wpe_hbm.at[pl.ds(p0, PW)], pos_v, spos)
    id_h = [pltpu.async_copy(ids_hbm.at[bb, pl.ds(p0, PW)], idx_v.at[bb],
                             sidx)
            for bb in range(B)]

    def start_gather(ch):
        id_h[ch].wait()
        return pltpu.async_copy(
            wte_hbm.at[idx_v.at[ch]], rows[ch % NBUF], sg[ch % NBUF])

    inflight = {0: start_gather(0), 1: start_gather(1)}
    pre.wait()

    store_h = [None, None, None]
    for ch in range(B):
        b = ch % NBUF
        if ch + 2 < B:
            nb = (ch + 2) % NBUF
            if store_h[nb] is not None:
                store_h[nb].wait()
                store_h[nb] = None
            inflight[ch + 2] = start_gather(ch + 2)
        inflight.pop(ch).wait()

        def add_row(r, carry):
            for j in range(NVEC):
                plsc.addupdate(rows[b].at[r, pl.ds(j * 16, 16)],
                               pos_v[r, pl.ds(j * 16, 16)])
            return carry

        lax.fori_loop(0, PW, add_row, 0)
        store_h[b] = pltpu.async_copy(
            rows[b], out_hbm.at[pl.ds(ch * S + p0, PW)], ss[b])
    for h in store_h:
        if h is not None:
            h.wait()


def kernel(input_ids, wte, wpe):
    out = _embed(input_ids.astype(jnp.int32), wte, wpe)
    return out.reshape(input_ids.shape + (wpe.shape[1],))
